# baseline (speedup 1.0000x reference)
import jax
import jax.numpy as jnp
from jax.experimental import pallas as pl
from jax.experimental.pallas import tpu as pltpu

N_USERS = 4096
X_DIM = 64
K_SUBS = 20000
R_DIM = 3

BM = 128
NM = N_USERS // BM


def _body(x_ref, s_ref, r_ref, o_ref):
    o_ref[:, :X_DIM] = x_ref[...]
    o_ref[:, X_DIM:] = jnp.dot(
        s_ref[...], r_ref[...], preferred_element_type=jnp.float32
    )


def kernel(x, S, R):
    return pl.pallas_call(
        _body,
        grid=(NM,),
        in_specs=[
            pl.BlockSpec((BM, X_DIM), lambda m: (m, 0)),
            pl.BlockSpec((BM, K_SUBS), lambda m: (m, 0)),
            pl.BlockSpec((K_SUBS, R_DIM), lambda m: (0, 0)),
        ],
        out_specs=pl.BlockSpec((BM, X_DIM + R_DIM), lambda m: (m, 0)),
        out_shape=jax.ShapeDtypeStruct((N_USERS, X_DIM + R_DIM), jnp.float32),
        compiler_params=pltpu.CompilerParams(
            dimension_semantics=("arbitrary",),
        ),
    )(x, S, R)


# speedup vs baseline: 1.0195x; 1.0195x over previous
import jax
import jax.numpy as jnp
from jax.experimental import pallas as pl
from jax.experimental.pallas import tpu as pltpu

N_USERS = 4096
X_DIM = 64
K_SUBS = 20000
R_DIM = 3

BM = 128
NM = N_USERS // BM


def _body(x_ref, s_ref, rt_ref, o_ref):
    o_ref[:, :X_DIM] = x_ref[...]
    s = s_ref[...]
    for j in range(R_DIM):
        rj = rt_ref[j : j + 1, :]
        o_ref[:, X_DIM + j : X_DIM + j + 1] = jnp.sum(
            s * rj, axis=1, keepdims=True
        )


def kernel(x, S, R):
    return pl.pallas_call(
        _body,
        grid=(NM,),
        in_specs=[
            pl.BlockSpec((BM, X_DIM), lambda m: (m, 0)),
            pl.BlockSpec((BM, K_SUBS), lambda m: (m, 0)),
            pl.BlockSpec((R_DIM, K_SUBS), lambda m: (0, 0)),
        ],
        out_specs=pl.BlockSpec((BM, X_DIM + R_DIM), lambda m: (m, 0)),
        out_shape=jax.ShapeDtypeStruct((N_USERS, X_DIM + R_DIM), jnp.float32),
        compiler_params=pltpu.CompilerParams(
            dimension_semantics=("arbitrary",),
        ),
    )(x, S, R.T)
